# Initial kernel scaffold; baseline (speedup 1.0000x reference)
import jax
import jax.numpy as jnp
from jax.experimental import pallas as pl


def kernel(pos, batch, pos_non_manifold, pos_non_manifold_batch, latents, variance, occupancies, W_in, b_in, W1, b1, W2, b2, W_out, b_out, dropout_rate):
    raise NotImplementedError("write your pallas kernel here")


# speedup vs baseline: 10.2558x; 10.2558x over previous
import functools

import jax
import jax.numpy as jnp
from jax import lax
from jax.experimental import pallas as pl
from jax.experimental.pallas import tpu as pltpu
from jax.experimental.pallas import tpu_sc as plsc

LATENT = 128
K = 16
MIN_VAR = 1e-3


def _tables_body(lat_ref, var_ref, posp_ref, post_ref, wlt_ref, wl2_ref,
                 wpt_ref, wp2_ref, tsrc_ref, ttgt_ref):
    f32 = jnp.float32
    lat = lat_ref[...]
    var = var_ref[...]
    posp = posp_ref[...]
    post = post_ref[...]
    wlt = wlt_ref[...]
    wl2 = wl2_ref[...]
    wpt = wpt_ref[...]
    wp2 = wp2_ref[...]
    am = jnp.dot(lat, wlt, preferred_element_type=f32) - jnp.dot(
        posp, wpt, preferred_element_type=f32)
    av = jnp.dot(var, wl2, preferred_element_type=f32) - jnp.dot(
        posp, wp2, preferred_element_type=f32)
    tsrc_ref[:, :LATENT] = am
    tsrc_ref[:, LATENT:] = av
    ttgt_ref[:, :LATENT] = jnp.dot(post, wpt, preferred_element_type=f32)
    ttgt_ref[:, LATENT:] = jnp.dot(post, wp2, preferred_element_type=f32)


def _make_tables(latents, variance, posp, postgt, wlt, wl2, wpt, wp2):
    n_src = latents.shape[0]
    n_tgt = postgt.shape[0]
    return pl.pallas_call(
        _tables_body,
        out_shape=(
            jax.ShapeDtypeStruct((n_src, 2 * LATENT), jnp.float32),
            jax.ShapeDtypeStruct((n_tgt, 2 * LATENT), jnp.float32),
        ),
    )(latents, variance, posp, postgt, wlt, wl2, wpt, wp2)


_KNN_TB = 256


def _knn_body(yt_ref, xt_ref, idx_ref):
    y = yt_ref[...]
    xt = xt_ref[...]
    n_src = xt.shape[1]
    tb = y.shape[0]
    yy = jnp.sum(y * y, axis=1, keepdims=True)
    xx = jnp.sum(xt * xt, axis=0, keepdims=True)
    dot = jnp.dot(y, xt, preferred_element_type=jnp.float32)
    d = (yy + xx) - 2.0 * dot
    colid = lax.broadcasted_iota(jnp.int32, (tb, n_src), 1)
    big = jnp.int32(2 ** 30)
    inf = jnp.float32(jnp.inf)
    cols = []
    for _ in range(K):
        m = jnp.min(d, axis=1, keepdims=True)
        sel = jnp.where(d <= m, colid, big)
        idx = jnp.min(sel, axis=1, keepdims=True)
        cols.append(idx)
        d = jnp.where(colid == idx, inf, d)
    idx_ref[...] = jnp.concatenate(cols, axis=1)


def _knn(postgt_pad, pos_t):
    n_tgt = postgt_pad.shape[0]
    grid = n_tgt // _KNN_TB
    return pl.pallas_call(
        _knn_body,
        grid=(grid,),
        in_specs=[
            pl.BlockSpec((_KNN_TB, 8), lambda i: (i, 0)),
            pl.BlockSpec(pos_t.shape, lambda i: (0, 0)),
        ],
        out_specs=pl.BlockSpec((_KNN_TB, K), lambda i: (i, 0)),
        out_shape=jax.ShapeDtypeStruct((n_tgt, K), jnp.int32),
    )(postgt_pad, pos_t)


_SC_CHUNK = 128


def _sc_gather(table, idx_flat):
    n_rows = idx_flat.shape[0]
    width = table.shape[1]
    info = plsc.get_sparse_core_info()
    nw = info.num_cores * info.num_subcores
    per_w = n_rows // nw
    n_chunks = per_w // _SC_CHUNK
    mesh = plsc.VectorSubcoreMesh(core_axis_name="c", subcore_axis_name="s")

    @functools.partial(
        pl.kernel,
        out_type=jax.ShapeDtypeStruct((n_rows, width), jnp.float32),
        mesh=mesh,
        scratch_types=[
            pltpu.VMEM((_SC_CHUNK,), jnp.int32),
            pltpu.VMEM((_SC_CHUNK, width), jnp.float32),
            pltpu.SemaphoreType.DMA,
        ],
    )
    def gather_k(table_hbm, idx_hbm, out_hbm, idx_v, rows_v, sem):
        wid = lax.axis_index("s") * info.num_cores + lax.axis_index("c")
        base_w = wid * per_w

        def chunk(c, _):
            base = base_w + c * _SC_CHUNK
            pltpu.sync_copy(idx_hbm.at[pl.ds(base, _SC_CHUNK)], idx_v)
            pltpu.async_copy(table_hbm.at[idx_v], rows_v, sem).wait()
            pltpu.sync_copy(rows_v, out_hbm.at[pl.ds(base, _SC_CHUNK)])
            return _

        lax.fori_loop(0, n_chunks, chunk, None)

    return gather_k(table, idx_flat)


_MLP_BLK = 2048
_MLP_TGT = _MLP_BLK // K

_SQRT_2PI = 2.5066282746310002
_SQRT_2 = 1.4142135623730951


def _adf_relu(mean, var):
    std = jnp.sqrt(jnp.clip(var, 1e-12, None))
    div = mean / std
    pdf = jnp.exp(-0.5 * div * div) / _SQRT_2PI
    cdf = 0.5 * (1.0 + lax.erf(div / _SQRT_2))
    out_mean = mean * cdf + std * pdf
    out_var = (mean * mean + var) * cdf + mean * std * pdf \
        - out_mean * out_mean
    return out_mean, out_var + MIN_VAR


def _mlp_body(g_ref, t_ref, occ_ref, w1t_ref, w12_ref, w2t_ref, w22_ref,
              wot_ref, wo2_ref, bin_ref, b1_ref, b2_ref, bo_ref,
              pred_ref, psum_ref):
    f32 = jnp.float32
    g = g_ref[...]
    t = t_ref[...]
    tgt, width = t.shape
    blk = g.shape[0]
    lat = LATENT
    t_rep = jnp.reshape(
        jnp.broadcast_to(t[:, None, :], (tgt, K, width)), (blk, width))
    mean = g[:, :lat] + t_rep[:, :lat] + bin_ref[...]
    var = g[:, lat:] + t_rep[:, lat:] + MIN_VAR

    mean, var = _adf_relu(mean, var)
    mean = jnp.dot(mean, w1t_ref[...], preferred_element_type=f32) \
        + b1_ref[...]
    var = jnp.dot(var, w12_ref[...], preferred_element_type=f32) + MIN_VAR

    mean, var = _adf_relu(mean, var)
    mean = jnp.dot(mean, w2t_ref[...], preferred_element_type=f32) \
        + b2_ref[...]
    var = jnp.dot(var, w22_ref[...], preferred_element_type=f32) + MIN_VAR

    out_m = jnp.dot(mean, wot_ref[...], preferred_element_type=f32) \
        + bo_ref[...]
    out_v = jnp.dot(var, wo2_ref[...], preferred_element_type=f32) + MIN_VAR

    z = out_m[:, 0:1]
    pred_ref[...] = z
    vsum = jnp.sum(jnp.abs(out_v[:, 0:1]))
    occ = occ_ref[...]
    loss = jnp.maximum(z, 0.0) - z * occ + jnp.log1p(jnp.exp(-jnp.abs(z)))
    lsum = jnp.sum(loss)
    psum_ref[...] = jnp.concatenate(
        [jnp.reshape(vsum, (1, 1, 1)), jnp.reshape(lsum, (1, 1, 1))], axis=2)


def _mlp(g, t_tgt, occ_gt, w1t, w12, w2t, w22, wot, wo2, b_in, b1, b2, b_o):
    n_rows = g.shape[0]
    grid = n_rows // _MLP_BLK
    full = lambda a: pl.BlockSpec(a.shape, lambda i: (0,) * a.ndim)
    pred, psum = pl.pallas_call(
        _mlp_body,
        grid=(grid,),
        in_specs=[
            pl.BlockSpec((_MLP_BLK, 2 * LATENT), lambda i: (i, 0)),
            pl.BlockSpec((_MLP_TGT, 2 * LATENT), lambda i: (i, 0)),
            pl.BlockSpec((_MLP_BLK, 1), lambda i: (i, 0)),
            full(w1t), full(w12), full(w2t), full(w22), full(wot), full(wo2),
            full(b_in), full(b1), full(b2), full(b_o),
        ],
        out_specs=(
            pl.BlockSpec((_MLP_BLK, 1), lambda i: (i, 0)),
            pl.BlockSpec((1, 1, 2), lambda i: (i, 0, 0)),
        ),
        out_shape=(
            jax.ShapeDtypeStruct((n_rows, 1), jnp.float32),
            jax.ShapeDtypeStruct((grid, 1, 2), jnp.float32),
        ),
    )(g, t_tgt, occ_gt, w1t, w12, w2t, w22, wot, wo2, b_in, b1, b2, b_o)
    return pred, psum


def kernel(pos, batch, pos_non_manifold, pos_non_manifold_batch, latents,
           variance, occupancies, W_in, b_in, W1, b1, W2, b2, W_out, b_out,
           dropout_rate):
    n_src = pos.shape[0]
    n_tgt = pos_non_manifold.shape[0]
    f32 = jnp.float32

    w_lat = W_in[:, :LATENT]
    w_pos = W_in[:, LATENT:]
    wlt = w_lat.T
    wl2 = (w_lat * w_lat).T
    wpt = jnp.concatenate([w_pos.T, jnp.zeros((5, LATENT), f32)], axis=0)
    wp2 = jnp.concatenate([(w_pos * w_pos).T, jnp.zeros((5, LATENT), f32)],
                          axis=0)
    posp = jnp.pad(pos, ((0, 0), (0, 5)))
    postp = jnp.pad(pos_non_manifold, ((0, 0), (0, 5)))
    pos_t = posp.T

    t_src, t_tgt = _make_tables(latents, variance, posp, postp,
                                wlt, wl2, wpt, wp2)
    idx = _knn(postp, pos_t)
    gathered = _sc_gather(t_src, idx.reshape(-1))

    occ_gt = jnp.broadcast_to(occupancies[:, None],
                              (n_tgt, K)).reshape(-1)

    w1t = W1.T
    w12 = (W1 * W1).T
    w2t = W2.T
    w22 = (W2 * W2).T
    wot = W_out.T
    wo2 = (W_out * W_out).T
    pred, psum = _mlp(gathered, t_tgt, occ_gt.reshape(-1, 1),
                      w1t, w12, w2t, w22, wot, wo2,
                      b_in.reshape(1, -1), b1.reshape(1, -1),
                      b2.reshape(1, -1), b_out.reshape(1, -1))

    n_pairs = n_tgt * K
    predictions = pred.reshape(n_pairs)
    aleatoric = jnp.sum(psum[:, 0, 0]) / n_pairs
    recons = jnp.sum(psum[:, 0, 1]) / n_pairs
    return (predictions, aleatoric, occ_gt, predictions, recons)


# speedup vs baseline: 11.6801x; 1.1389x over previous
import functools

import jax
import jax.numpy as jnp
from jax import lax
from jax.experimental import pallas as pl
from jax.experimental.pallas import tpu as pltpu
from jax.experimental.pallas import tpu_sc as plsc

LATENT = 128
K = 16
MIN_VAR = 1e-3


def _tables_body(lat_ref, var_ref, posp_ref, post_ref, wlt_ref, wl2_ref,
                 wpt_ref, wp2_ref, tsrc_ref, ttgt_ref):
    f32 = jnp.float32
    lat = lat_ref[...]
    var = var_ref[...]
    posp = posp_ref[...]
    post = post_ref[...]
    wlt = wlt_ref[...]
    wl2 = wl2_ref[...]
    wpt = wpt_ref[...]
    wp2 = wp2_ref[...]
    am = jnp.dot(lat, wlt, preferred_element_type=f32) - jnp.dot(
        posp, wpt, preferred_element_type=f32)
    av = jnp.dot(var, wl2, preferred_element_type=f32) - jnp.dot(
        posp, wp2, preferred_element_type=f32)
    tsrc_ref[:, :LATENT] = am
    tsrc_ref[:, LATENT:] = av
    ttgt_ref[:, :LATENT] = jnp.dot(post, wpt, preferred_element_type=f32)
    ttgt_ref[:, LATENT:] = jnp.dot(post, wp2, preferred_element_type=f32)


def _make_tables(latents, variance, posp, postgt, wlt, wl2, wpt, wp2):
    n_src = latents.shape[0]
    n_tgt = postgt.shape[0]
    return pl.pallas_call(
        _tables_body,
        out_shape=(
            jax.ShapeDtypeStruct((n_src, 2 * LATENT), jnp.float32),
            jax.ShapeDtypeStruct((n_tgt, 2 * LATENT), jnp.float32),
        ),
    )(latents, variance, posp, postgt, wlt, wl2, wpt, wp2)


_KNN_TB = 256


def _knn_body(yt_ref, xt_ref, idx_ref):
    y = yt_ref[...]
    xt = xt_ref[...]
    n_src = xt.shape[1]
    tb = y.shape[0]
    yy = jnp.sum(y * y, axis=1, keepdims=True)
    xx = jnp.sum(xt * xt, axis=0, keepdims=True)
    dot = jnp.dot(y, xt, preferred_element_type=jnp.float32)
    d = (yy + xx) - 2.0 * dot
    colf = lax.broadcasted_iota(jnp.int32, (tb, n_src), 1).astype(jnp.float32)
    big = jnp.float32(1e30)
    inf = jnp.float32(jnp.inf)
    cols = []
    for _ in range(K):
        m = jnp.min(d, axis=1, keepdims=True)
        sel = jnp.where(d <= m, colf, big)
        idxf = jnp.min(sel, axis=1, keepdims=True)
        cols.append(idxf)
        d = jnp.where(colf == idxf, inf, d)
    idx_ref[...] = jnp.concatenate(cols, axis=1).astype(jnp.int32)


def _knn(postgt_pad, pos_t):
    n_tgt = postgt_pad.shape[0]
    grid = n_tgt // _KNN_TB
    return pl.pallas_call(
        _knn_body,
        grid=(grid,),
        in_specs=[
            pl.BlockSpec((_KNN_TB, 8), lambda i: (i, 0)),
            pl.BlockSpec(pos_t.shape, lambda i: (0, 0)),
        ],
        out_specs=pl.BlockSpec((_KNN_TB, K), lambda i: (i, 0)),
        out_shape=jax.ShapeDtypeStruct((n_tgt, K), jnp.int32),
    )(postgt_pad, pos_t)


_SC_CHUNK = 128


def _sc_gather(table, idx_flat):
    n_rows = idx_flat.shape[0]
    width = table.shape[1]
    info = plsc.get_sparse_core_info()
    nw = info.num_cores * info.num_subcores
    per_w = n_rows // nw
    n_chunks = per_w // _SC_CHUNK
    mesh = plsc.VectorSubcoreMesh(core_axis_name="c", subcore_axis_name="s")

    @functools.partial(
        pl.kernel,
        out_type=jax.ShapeDtypeStruct((n_rows, width), jnp.float32),
        mesh=mesh,
        scratch_types=[
            pltpu.VMEM((_SC_CHUNK,), jnp.int32),
            pltpu.VMEM((_SC_CHUNK, width), jnp.float32),
            pltpu.SemaphoreType.DMA,
        ],
    )
    def gather_k(table_hbm, idx_hbm, out_hbm, idx_v, rows_v, sem):
        wid = lax.axis_index("s") * info.num_cores + lax.axis_index("c")
        base_w = wid * per_w

        def chunk(c, _):
            base = base_w + c * _SC_CHUNK
            pltpu.sync_copy(idx_hbm.at[pl.ds(base, _SC_CHUNK)], idx_v)
            pltpu.async_copy(table_hbm.at[idx_v], rows_v, sem).wait()
            pltpu.sync_copy(rows_v, out_hbm.at[pl.ds(base, _SC_CHUNK)])
            return _

        lax.fori_loop(0, n_chunks, chunk, None)

    return gather_k(table, idx_flat)


_MLP_BLK = 2048
_MLP_TGT = _MLP_BLK // K

_SQRT_2PI = 2.5066282746310002
_SQRT_2 = 1.4142135623730951


def _adf_relu(mean, var):
    std = jnp.sqrt(jnp.clip(var, 1e-12, None))
    div = mean / std
    pdf = jnp.exp(-0.5 * div * div) / _SQRT_2PI
    cdf = 0.5 * (1.0 + lax.erf(div / _SQRT_2))
    out_mean = mean * cdf + std * pdf
    out_var = (mean * mean + var) * cdf + mean * std * pdf \
        - out_mean * out_mean
    return out_mean, out_var + MIN_VAR


def _mlp_body(g_ref, t_ref, occ_ref, w1t_ref, w12_ref, w2t_ref, w22_ref,
              wot_ref, wo2_ref, bin_ref, b1_ref, b2_ref, bo_ref,
              pred_ref, psum_ref):
    f32 = jnp.float32
    g = g_ref[...]
    t = t_ref[...]
    tgt, width = t.shape
    blk = g.shape[0]
    lat = LATENT
    t_rep = jnp.reshape(
        jnp.broadcast_to(t[:, None, :], (tgt, K, width)), (blk, width))
    mean = g[:, :lat] + t_rep[:, :lat] + bin_ref[...]
    var = g[:, lat:] + t_rep[:, lat:] + MIN_VAR

    mean, var = _adf_relu(mean, var)
    mean = jnp.dot(mean, w1t_ref[...], preferred_element_type=f32) \
        + b1_ref[...]
    var = jnp.dot(var, w12_ref[...], preferred_element_type=f32) + MIN_VAR

    mean, var = _adf_relu(mean, var)
    mean = jnp.dot(mean, w2t_ref[...], preferred_element_type=f32) \
        + b2_ref[...]
    var = jnp.dot(var, w22_ref[...], preferred_element_type=f32) + MIN_VAR

    out_m = jnp.dot(mean, wot_ref[...], preferred_element_type=f32) \
        + bo_ref[...]
    out_v = jnp.dot(var, wo2_ref[...], preferred_element_type=f32) + MIN_VAR

    z = out_m[:, 0:1]
    pred_ref[...] = z
    vsum = jnp.sum(jnp.abs(out_v[:, 0:1]))
    occ = occ_ref[...]
    loss = jnp.maximum(z, 0.0) - z * occ + jnp.log1p(jnp.exp(-jnp.abs(z)))
    lsum = jnp.sum(loss)
    psum_ref[...] = jnp.concatenate(
        [jnp.reshape(vsum, (1, 1, 1)), jnp.reshape(lsum, (1, 1, 1))], axis=2)


def _mlp(g, t_tgt, occ_gt, w1t, w12, w2t, w22, wot, wo2, b_in, b1, b2, b_o):
    n_rows = g.shape[0]
    grid = n_rows // _MLP_BLK
    full = lambda a: pl.BlockSpec(a.shape, lambda i: (0,) * a.ndim)
    pred, psum = pl.pallas_call(
        _mlp_body,
        grid=(grid,),
        in_specs=[
            pl.BlockSpec((_MLP_BLK, 2 * LATENT), lambda i: (i, 0)),
            pl.BlockSpec((_MLP_TGT, 2 * LATENT), lambda i: (i, 0)),
            pl.BlockSpec((_MLP_BLK, 1), lambda i: (i, 0)),
            full(w1t), full(w12), full(w2t), full(w22), full(wot), full(wo2),
            full(b_in), full(b1), full(b2), full(b_o),
        ],
        out_specs=(
            pl.BlockSpec((_MLP_BLK, 1), lambda i: (i, 0)),
            pl.BlockSpec((1, 1, 2), lambda i: (i, 0, 0)),
        ),
        out_shape=(
            jax.ShapeDtypeStruct((n_rows, 1), jnp.float32),
            jax.ShapeDtypeStruct((grid, 1, 2), jnp.float32),
        ),
    )(g, t_tgt, occ_gt, w1t, w12, w2t, w22, wot, wo2, b_in, b1, b2, b_o)
    return pred, psum


def kernel(pos, batch, pos_non_manifold, pos_non_manifold_batch, latents,
           variance, occupancies, W_in, b_in, W1, b1, W2, b2, W_out, b_out,
           dropout_rate):
    n_src = pos.shape[0]
    n_tgt = pos_non_manifold.shape[0]
    f32 = jnp.float32

    w_lat = W_in[:, :LATENT]
    w_pos = W_in[:, LATENT:]
    wlt = w_lat.T
    wl2 = (w_lat * w_lat).T
    wpt = jnp.concatenate([w_pos.T, jnp.zeros((5, LATENT), f32)], axis=0)
    wp2 = jnp.concatenate([(w_pos * w_pos).T, jnp.zeros((5, LATENT), f32)],
                          axis=0)
    posp = jnp.pad(pos, ((0, 0), (0, 5)))
    postp = jnp.pad(pos_non_manifold, ((0, 0), (0, 5)))
    pos_t = posp.T

    t_src, t_tgt = _make_tables(latents, variance, posp, postp,
                                wlt, wl2, wpt, wp2)
    idx = _knn(postp, pos_t)
    gathered = _sc_gather(t_src, idx.reshape(-1))

    occ_gt = jnp.broadcast_to(occupancies[:, None],
                              (n_tgt, K)).reshape(-1)

    w1t = W1.T
    w12 = (W1 * W1).T
    w2t = W2.T
    w22 = (W2 * W2).T
    wot = W_out.T
    wo2 = (W_out * W_out).T
    pred, psum = _mlp(gathered, t_tgt, occ_gt.reshape(-1, 1),
                      w1t, w12, w2t, w22, wot, wo2,
                      b_in.reshape(1, -1), b1.reshape(1, -1),
                      b2.reshape(1, -1), b_out.reshape(1, -1))

    n_pairs = n_tgt * K
    predictions = pred.reshape(n_pairs)
    aleatoric = jnp.sum(psum[:, 0, 0]) / n_pairs
    recons = jnp.sum(psum[:, 0, 1]) / n_pairs
    return (predictions, aleatoric, occ_gt, predictions, recons)


# speedup vs baseline: 12.0817x; 1.0344x over previous
import functools

import jax
import jax.numpy as jnp
from jax import lax
from jax.experimental import pallas as pl
from jax.experimental.pallas import tpu as pltpu
from jax.experimental.pallas import tpu_sc as plsc

LATENT = 128
K = 16
MIN_VAR = 1e-3


def _tables_body(lat_ref, var_ref, posp_ref, post_ref, wlt_ref, wl2_ref,
                 wpt_ref, wp2_ref, tsrc_ref, ttgt_ref):
    f32 = jnp.float32
    lat = lat_ref[...]
    var = var_ref[...]
    posp = posp_ref[...]
    post = post_ref[...]
    wlt = wlt_ref[...]
    wl2 = wl2_ref[...]
    wpt = wpt_ref[...]
    wp2 = wp2_ref[...]
    am = jnp.dot(lat, wlt, preferred_element_type=f32) - jnp.dot(
        posp, wpt, preferred_element_type=f32)
    av = jnp.dot(var, wl2, preferred_element_type=f32) - jnp.dot(
        posp, wp2, preferred_element_type=f32)
    tsrc_ref[:, :LATENT] = am
    tsrc_ref[:, LATENT:] = av
    ttgt_ref[:, :LATENT] = jnp.dot(post, wpt, preferred_element_type=f32)
    ttgt_ref[:, LATENT:] = jnp.dot(post, wp2, preferred_element_type=f32)


def _make_tables(latents, variance, posp, postgt, wlt, wl2, wpt, wp2):
    n_src = latents.shape[0]
    n_tgt = postgt.shape[0]
    return pl.pallas_call(
        _tables_body,
        out_shape=(
            jax.ShapeDtypeStruct((n_src, 2 * LATENT), jnp.float32),
            jax.ShapeDtypeStruct((n_tgt, 2 * LATENT), jnp.float32),
        ),
    )(latents, variance, posp, postgt, wlt, wl2, wpt, wp2)


_KNN_TB = 256


def _knn_body(yt_ref, xt_ref, idx_ref):
    y = yt_ref[...]
    xt = xt_ref[...]
    n_src = xt.shape[1]
    tb = y.shape[0]
    yy = jnp.sum(y * y, axis=1, keepdims=True)
    xx = jnp.sum(xt * xt, axis=0, keepdims=True)
    dot = jnp.dot(y, xt, preferred_element_type=jnp.float32)
    d = (yy + xx) - 2.0 * dot
    colf = lax.broadcasted_iota(jnp.int32, (tb, n_src), 1).astype(jnp.float32)
    big = jnp.float32(1e30)
    inf = jnp.float32(jnp.inf)
    cols = []
    for _ in range(K):
        m = jnp.min(d, axis=1, keepdims=True)
        sel = jnp.where(d <= m, colf, big)
        idxf = jnp.min(sel, axis=1, keepdims=True)
        cols.append(idxf)
        d = jnp.where(colf == idxf, inf, d)
    idx_ref[...] = jnp.concatenate(cols, axis=1).astype(jnp.int32)


def _knn(postgt_pad, pos_t):
    n_tgt = postgt_pad.shape[0]
    grid = n_tgt // _KNN_TB
    return pl.pallas_call(
        _knn_body,
        grid=(grid,),
        in_specs=[
            pl.BlockSpec((_KNN_TB, 8), lambda i: (i, 0)),
            pl.BlockSpec(pos_t.shape, lambda i: (0, 0)),
        ],
        out_specs=pl.BlockSpec((_KNN_TB, K), lambda i: (i, 0)),
        out_shape=jax.ShapeDtypeStruct((n_tgt, K), jnp.int32),
    )(postgt_pad, pos_t)


_SC_CHUNK = 128


def _sc_gather(table, idx_flat):
    n_rows = idx_flat.shape[0]
    width = table.shape[1]
    info = plsc.get_sparse_core_info()
    nw = info.num_cores * info.num_subcores
    per_w = n_rows // nw
    n_chunks = per_w // _SC_CHUNK
    mesh = plsc.VectorSubcoreMesh(core_axis_name="c", subcore_axis_name="s")

    @functools.partial(
        pl.kernel,
        out_type=jax.ShapeDtypeStruct((n_rows, width), jnp.float32),
        mesh=mesh,
        scratch_types=[
            pltpu.VMEM((_SC_CHUNK,), jnp.int32),
            pltpu.VMEM((_SC_CHUNK, width), jnp.float32),
            pltpu.SemaphoreType.DMA,
        ],
    )
    def gather_k(table_hbm, idx_hbm, out_hbm, idx_v, rows_v, sem):
        wid = lax.axis_index("s") * info.num_cores + lax.axis_index("c")
        base_w = wid * per_w

        def chunk(c, _):
            base = base_w + c * _SC_CHUNK
            pltpu.sync_copy(idx_hbm.at[pl.ds(base, _SC_CHUNK)], idx_v)
            pltpu.async_copy(table_hbm.at[idx_v], rows_v, sem).wait()
            pltpu.sync_copy(rows_v, out_hbm.at[pl.ds(base, _SC_CHUNK)])
            return _

        lax.fori_loop(0, n_chunks, chunk, None)

    return gather_k(table, idx_flat)


_MLP_BLK = 2048
_MLP_TGT = _MLP_BLK // K

_SQRT_2PI = 2.5066282746310002
_SQRT_2 = 1.4142135623730951


def _adf_relu(mean, var):
    std = jnp.sqrt(jnp.clip(var, 1e-12, None))
    div = mean / std
    pdf = jnp.exp(-0.5 * div * div) / _SQRT_2PI
    cdf = 0.5 * (1.0 + lax.erf(div / _SQRT_2))
    out_mean = mean * cdf + std * pdf
    out_var = (mean * mean + var) * cdf + mean * std * pdf \
        - out_mean * out_mean
    return out_mean, out_var + MIN_VAR


def _mlp_body(g_ref, t_ref, occ_ref, w1t_ref, w12_ref, w2t_ref, w22_ref,
              wot_ref, wo2_ref, bin_ref, b1_ref, b2_ref, bo_ref,
              pred_ref, psum_ref):
    f32 = jnp.float32
    g = g_ref[...]
    t = t_ref[...]
    tgt, width = t.shape
    blk = g.shape[0]
    lat = LATENT
    t_rep = jnp.reshape(
        jnp.broadcast_to(t[:, None, :], (tgt, K, width)), (blk, width))
    mean = g[:, :lat] + t_rep[:, :lat] + bin_ref[...]
    var = g[:, lat:] + t_rep[:, lat:] + MIN_VAR

    mean, var = _adf_relu(mean, var)
    mean = jnp.dot(mean, w1t_ref[...], preferred_element_type=f32) \
        + b1_ref[...]
    var = jnp.dot(var, w12_ref[...], preferred_element_type=f32) + MIN_VAR

    mean, var = _adf_relu(mean, var)
    mean = jnp.dot(mean, w2t_ref[...], preferred_element_type=f32) \
        + b2_ref[...]
    var = jnp.dot(var, w22_ref[...], preferred_element_type=f32) + MIN_VAR

    out_m = jnp.dot(mean, wot_ref[...], preferred_element_type=f32) \
        + bo_ref[...]
    out_v = jnp.dot(var, wo2_ref[...], preferred_element_type=f32) + MIN_VAR

    z = out_m[:, 0:1]
    pred_ref[...] = z
    vsum = jnp.sum(jnp.abs(out_v[:, 0:1]))
    occ = occ_ref[...]
    loss = jnp.maximum(z, 0.0) - z * occ + jnp.log1p(jnp.exp(-jnp.abs(z)))
    lsum = jnp.sum(loss)
    psum_ref[...] = jnp.concatenate(
        [jnp.reshape(vsum, (1, 1, 1)), jnp.reshape(lsum, (1, 1, 1))], axis=2)


def _mlp(g, t_tgt, occ_gt, w1t, w12, w2t, w22, wot, wo2, b_in, b1, b2, b_o):
    n_rows = g.shape[0]
    grid = n_rows // _MLP_BLK
    full = lambda a: pl.BlockSpec(a.shape, lambda i: (0,) * a.ndim)
    pred, psum = pl.pallas_call(
        _mlp_body,
        grid=(grid,),
        in_specs=[
            pl.BlockSpec((_MLP_BLK, 2 * LATENT), lambda i: (i, 0)),
            pl.BlockSpec((_MLP_TGT, 2 * LATENT), lambda i: (i, 0)),
            pl.BlockSpec((_MLP_BLK, 1), lambda i: (i, 0)),
            full(w1t), full(w12), full(w2t), full(w22), full(wot), full(wo2),
            full(b_in), full(b1), full(b2), full(b_o),
        ],
        out_specs=(
            pl.BlockSpec((_MLP_BLK, 1), lambda i: (i, 0)),
            pl.BlockSpec((1, 1, 2), lambda i: (i, 0, 0)),
        ),
        out_shape=(
            jax.ShapeDtypeStruct((n_rows, 1), jnp.float32),
            jax.ShapeDtypeStruct((grid, 1, 2), jnp.float32),
        ),
    )(g, t_tgt, occ_gt, w1t, w12, w2t, w22, wot, wo2, b_in, b1, b2, b_o)
    return pred, psum


def kernel(pos, batch, pos_non_manifold, pos_non_manifold_batch, latents,
           variance, occupancies, W_in, b_in, W1, b1, W2, b2, W_out, b_out,
           dropout_rate):
    n_src = pos.shape[0]
    n_tgt = pos_non_manifold.shape[0]
    f32 = jnp.float32

    w_lat = W_in[:, :LATENT]
    w_pos = W_in[:, LATENT:]
    wlt = w_lat.T
    wl2 = (w_lat * w_lat).T
    wpt = jnp.concatenate([w_pos.T, jnp.zeros((5, LATENT), f32)], axis=0)
    wp2 = jnp.concatenate([(w_pos * w_pos).T, jnp.zeros((5, LATENT), f32)],
                          axis=0)
    posp = jnp.pad(pos, ((0, 0), (0, 5)))
    postp = jnp.pad(pos_non_manifold, ((0, 0), (0, 5)))
    pos_t = posp.T

    t_src, t_tgt = _make_tables(latents, variance, posp, postp,
                                wlt, wl2, wpt, wp2)

    occ_gt = jnp.broadcast_to(occupancies[:, None],
                              (n_tgt, K)).reshape(-1)

    w1t = W1.T
    w12 = (W1 * W1).T
    w2t = W2.T
    w22 = (W2 * W2).T
    wot = W_out.T
    wo2 = (W_out * W_out).T
    weights = (w1t, w12, w2t, w22, wot, wo2, b_in.reshape(1, -1),
               b1.reshape(1, -1), b2.reshape(1, -1), b_out.reshape(1, -1))

    h = n_tgt // 2
    idx_a = _knn(postp[:h], pos_t)
    g_a = _sc_gather(t_src, idx_a.reshape(-1))
    idx_b = _knn(postp[h:], pos_t)
    g_b = _sc_gather(t_src, idx_b.reshape(-1))
    pred_a, psum_a = _mlp(g_a, t_tgt[:h], occ_gt[:h * K].reshape(-1, 1),
                          *weights)
    pred_b, psum_b = _mlp(g_b, t_tgt[h:], occ_gt[h * K:].reshape(-1, 1),
                          *weights)

    n_pairs = n_tgt * K
    predictions = jnp.concatenate([pred_a, pred_b]).reshape(n_pairs)
    aleatoric = (jnp.sum(psum_a[:, 0, 0]) + jnp.sum(psum_b[:, 0, 0])) / n_pairs
    recons = (jnp.sum(psum_a[:, 0, 1]) + jnp.sum(psum_b[:, 0, 1])) / n_pairs
    return (predictions, aleatoric, occ_gt, predictions, recons)


# speedup vs baseline: 12.0838x; 1.0002x over previous
import functools

import jax
import jax.numpy as jnp
from jax import lax
from jax.experimental import pallas as pl
from jax.experimental.pallas import tpu as pltpu
from jax.experimental.pallas import tpu_sc as plsc

LATENT = 128
K = 16
MIN_VAR = 1e-3


def _tables_body(lat_ref, var_ref, posp_ref, post_ref, wlt_ref, wl2_ref,
                 wpt_ref, wp2_ref, tsrc_ref, ttgt_ref):
    f32 = jnp.float32
    lat = lat_ref[...]
    var = var_ref[...]
    posp = posp_ref[...]
    post = post_ref[...]
    wlt = wlt_ref[...]
    wl2 = wl2_ref[...]
    wpt = wpt_ref[...]
    wp2 = wp2_ref[...]
    am = jnp.dot(lat, wlt, preferred_element_type=f32) - jnp.dot(
        posp, wpt, preferred_element_type=f32)
    av = jnp.dot(var, wl2, preferred_element_type=f32) - jnp.dot(
        posp, wp2, preferred_element_type=f32)
    tsrc_ref[:, :LATENT] = am
    tsrc_ref[:, LATENT:] = av
    ttgt_ref[:, :LATENT] = jnp.dot(post, wpt, preferred_element_type=f32)
    ttgt_ref[:, LATENT:] = jnp.dot(post, wp2, preferred_element_type=f32)


def _make_tables(latents, variance, posp, postgt, wlt, wl2, wpt, wp2):
    n_src = latents.shape[0]
    n_tgt = postgt.shape[0]
    return pl.pallas_call(
        _tables_body,
        out_shape=(
            jax.ShapeDtypeStruct((n_src, 2 * LATENT), jnp.float32),
            jax.ShapeDtypeStruct((n_tgt, 2 * LATENT), jnp.float32),
        ),
    )(latents, variance, posp, postgt, wlt, wl2, wpt, wp2)


_KNN_TB = 256


def _knn_body(yt_ref, xt_ref, idx_ref):
    y = yt_ref[...]
    xt = xt_ref[...]
    n_src = xt.shape[1]
    tb = y.shape[0]
    yy = jnp.sum(y * y, axis=1, keepdims=True)
    xx = jnp.sum(xt * xt, axis=0, keepdims=True)
    dot = jnp.dot(y, xt, preferred_element_type=jnp.float32)
    d = (yy + xx) - 2.0 * dot
    colf = lax.broadcasted_iota(jnp.int32, (tb, n_src), 1).astype(jnp.float32)
    big = jnp.float32(1e30)
    inf = jnp.float32(jnp.inf)
    cols = []
    for _ in range(K):
        m = jnp.min(d, axis=1, keepdims=True)
        sel = jnp.where(d <= m, colf, big)
        idxf = jnp.min(sel, axis=1, keepdims=True)
        cols.append(idxf)
        d = jnp.where(colf == idxf, inf, d)
    idx_ref[...] = jnp.concatenate(cols, axis=1).astype(jnp.int32)


def _knn(postgt_pad, pos_t):
    n_tgt = postgt_pad.shape[0]
    grid = n_tgt // _KNN_TB
    return pl.pallas_call(
        _knn_body,
        grid=(grid,),
        in_specs=[
            pl.BlockSpec((_KNN_TB, 8), lambda i: (i, 0)),
            pl.BlockSpec(pos_t.shape, lambda i: (0, 0)),
        ],
        out_specs=pl.BlockSpec((_KNN_TB, K), lambda i: (i, 0)),
        out_shape=jax.ShapeDtypeStruct((n_tgt, K), jnp.int32),
    )(postgt_pad, pos_t)


_SC_CHUNK = 128


def _sc_gather(table, idx_flat):
    n_rows = idx_flat.shape[0]
    width = table.shape[1]
    info = plsc.get_sparse_core_info()
    nw = info.num_cores * info.num_subcores
    per_w = n_rows // nw
    n_chunks = per_w // _SC_CHUNK
    mesh = plsc.VectorSubcoreMesh(core_axis_name="c", subcore_axis_name="s")

    @functools.partial(
        pl.kernel,
        out_type=jax.ShapeDtypeStruct((n_rows, width), jnp.float32),
        mesh=mesh,
        scratch_types=[
            pltpu.VMEM((_SC_CHUNK,), jnp.int32),
            pltpu.VMEM((_SC_CHUNK, width), jnp.float32),
            pltpu.SemaphoreType.DMA,
        ],
        compiler_params=pltpu.CompilerParams(use_tc_tiling_on_sc=True),
    )
    def gather_k(table_hbm, idx_hbm, out_hbm, idx_v, rows_v, sem):
        wid = lax.axis_index("s") * info.num_cores + lax.axis_index("c")
        base_w = wid * per_w

        def chunk(c, _):
            base = base_w + c * _SC_CHUNK
            pltpu.sync_copy(idx_hbm.at[pl.ds(base, _SC_CHUNK)], idx_v)
            pltpu.async_copy(table_hbm.at[idx_v], rows_v, sem).wait()
            pltpu.sync_copy(rows_v, out_hbm.at[pl.ds(base, _SC_CHUNK)])
            return _

        lax.fori_loop(0, n_chunks, chunk, None)

    return gather_k(table, idx_flat)


_MLP_BLK = 2048
_MLP_TGT = _MLP_BLK // K

_SQRT_2PI = 2.5066282746310002
_SQRT_2 = 1.4142135623730951


def _adf_relu(mean, var):
    std = jnp.sqrt(jnp.clip(var, 1e-12, None))
    div = mean / std
    pdf = jnp.exp(-0.5 * div * div) / _SQRT_2PI
    cdf = 0.5 * (1.0 + lax.erf(div / _SQRT_2))
    out_mean = mean * cdf + std * pdf
    out_var = (mean * mean + var) * cdf + mean * std * pdf \
        - out_mean * out_mean
    return out_mean, out_var + MIN_VAR


def _mlp_body(g_ref, t_ref, occ_ref, w1t_ref, w12_ref, w2t_ref, w22_ref,
              wot_ref, wo2_ref, bin_ref, b1_ref, b2_ref, bo_ref,
              pred_ref, psum_ref):
    f32 = jnp.float32
    g = g_ref[...]
    t = t_ref[...]
    tgt, width = t.shape
    blk = g.shape[0]
    lat = LATENT
    t_rep = jnp.reshape(
        jnp.broadcast_to(t[:, None, :], (tgt, K, width)), (blk, width))
    mean = g[:, :lat] + t_rep[:, :lat] + bin_ref[...]
    var = g[:, lat:] + t_rep[:, lat:] + MIN_VAR

    mean, var = _adf_relu(mean, var)
    mean = jnp.dot(mean, w1t_ref[...], preferred_element_type=f32) \
        + b1_ref[...]
    var = jnp.dot(var, w12_ref[...], preferred_element_type=f32) + MIN_VAR

    mean, var = _adf_relu(mean, var)
    mean = jnp.dot(mean, w2t_ref[...], preferred_element_type=f32) \
        + b2_ref[...]
    var = jnp.dot(var, w22_ref[...], preferred_element_type=f32) + MIN_VAR

    out_m = jnp.dot(mean, wot_ref[...], preferred_element_type=f32) \
        + bo_ref[...]
    out_v = jnp.dot(var, wo2_ref[...], preferred_element_type=f32) + MIN_VAR

    z = out_m[:, 0:1]
    pred_ref[...] = z
    vsum = jnp.sum(jnp.abs(out_v[:, 0:1]))
    occ = occ_ref[...]
    loss = jnp.maximum(z, 0.0) - z * occ + jnp.log1p(jnp.exp(-jnp.abs(z)))
    lsum = jnp.sum(loss)
    psum_ref[...] = jnp.concatenate(
        [jnp.reshape(vsum, (1, 1, 1)), jnp.reshape(lsum, (1, 1, 1))], axis=2)


def _mlp(g, t_tgt, occ_gt, w1t, w12, w2t, w22, wot, wo2, b_in, b1, b2, b_o):
    n_rows = g.shape[0]
    grid = n_rows // _MLP_BLK
    full = lambda a: pl.BlockSpec(a.shape, lambda i: (0,) * a.ndim)
    pred, psum = pl.pallas_call(
        _mlp_body,
        grid=(grid,),
        in_specs=[
            pl.BlockSpec((_MLP_BLK, 2 * LATENT), lambda i: (i, 0)),
            pl.BlockSpec((_MLP_TGT, 2 * LATENT), lambda i: (i, 0)),
            pl.BlockSpec((_MLP_BLK, 1), lambda i: (i, 0)),
            full(w1t), full(w12), full(w2t), full(w22), full(wot), full(wo2),
            full(b_in), full(b1), full(b2), full(b_o),
        ],
        out_specs=(
            pl.BlockSpec((_MLP_BLK, 1), lambda i: (i, 0)),
            pl.BlockSpec((1, 1, 2), lambda i: (i, 0, 0)),
        ),
        out_shape=(
            jax.ShapeDtypeStruct((n_rows, 1), jnp.float32),
            jax.ShapeDtypeStruct((grid, 1, 2), jnp.float32),
        ),
    )(g, t_tgt, occ_gt, w1t, w12, w2t, w22, wot, wo2, b_in, b1, b2, b_o)
    return pred, psum


def kernel(pos, batch, pos_non_manifold, pos_non_manifold_batch, latents,
           variance, occupancies, W_in, b_in, W1, b1, W2, b2, W_out, b_out,
           dropout_rate):
    n_src = pos.shape[0]
    n_tgt = pos_non_manifold.shape[0]
    f32 = jnp.float32

    w_lat = W_in[:, :LATENT]
    w_pos = W_in[:, LATENT:]
    wlt = w_lat.T
    wl2 = (w_lat * w_lat).T
    wpt = jnp.concatenate([w_pos.T, jnp.zeros((5, LATENT), f32)], axis=0)
    wp2 = jnp.concatenate([(w_pos * w_pos).T, jnp.zeros((5, LATENT), f32)],
                          axis=0)
    posp = jnp.pad(pos, ((0, 0), (0, 5)))
    postp = jnp.pad(pos_non_manifold, ((0, 0), (0, 5)))
    pos_t = posp.T

    t_src, t_tgt = _make_tables(latents, variance, posp, postp,
                                wlt, wl2, wpt, wp2)

    occ_gt = jnp.broadcast_to(occupancies[:, None],
                              (n_tgt, K)).reshape(-1)

    w1t = W1.T
    w12 = (W1 * W1).T
    w2t = W2.T
    w22 = (W2 * W2).T
    wot = W_out.T
    wo2 = (W_out * W_out).T
    weights = (w1t, w12, w2t, w22, wot, wo2, b_in.reshape(1, -1),
               b1.reshape(1, -1), b2.reshape(1, -1), b_out.reshape(1, -1))

    h = n_tgt // 2
    idx_a = _knn(postp[:h], pos_t)
    g_a = _sc_gather(t_src, idx_a.reshape(-1))
    idx_b = _knn(postp[h:], pos_t)
    g_b = _sc_gather(t_src, idx_b.reshape(-1))
    pred_a, psum_a = _mlp(g_a, t_tgt[:h], occ_gt[:h * K].reshape(-1, 1),
                          *weights)
    pred_b, psum_b = _mlp(g_b, t_tgt[h:], occ_gt[h * K:].reshape(-1, 1),
                          *weights)

    n_pairs = n_tgt * K
    predictions = jnp.concatenate([pred_a, pred_b]).reshape(n_pairs)
    aleatoric = (jnp.sum(psum_a[:, 0, 0]) + jnp.sum(psum_b[:, 0, 0])) / n_pairs
    recons = (jnp.sum(psum_a[:, 0, 1]) + jnp.sum(psum_b[:, 0, 1])) / n_pairs
    return (predictions, aleatoric, occ_gt, predictions, recons)


# speedup vs baseline: 13.4335x; 1.1117x over previous
import functools

import jax
import jax.numpy as jnp
from jax import lax
from jax.experimental import pallas as pl
from jax.experimental.pallas import tpu as pltpu
from jax.experimental.pallas import tpu_sc as plsc

LATENT = 128
K = 16
MIN_VAR = 1e-3


def _tables_body(lat_ref, var_ref, posp_ref, post_ref, wlt_ref, wl2_ref,
                 wpt_ref, wp2_ref, tsrc_ref, ttgt_ref):
    f32 = jnp.float32
    lat = lat_ref[...]
    var = var_ref[...]
    posp = posp_ref[...]
    post = post_ref[...]
    wlt = wlt_ref[...]
    wl2 = wl2_ref[...]
    wpt = wpt_ref[...]
    wp2 = wp2_ref[...]
    am = jnp.dot(lat, wlt, preferred_element_type=f32) - jnp.dot(
        posp, wpt, preferred_element_type=f32)
    av = jnp.dot(var, wl2, preferred_element_type=f32) - jnp.dot(
        posp, wp2, preferred_element_type=f32)
    tsrc_ref[:, :LATENT] = am
    tsrc_ref[:, LATENT:] = av
    ttgt_ref[:, :LATENT] = jnp.dot(post, wpt, preferred_element_type=f32)
    ttgt_ref[:, LATENT:] = jnp.dot(post, wp2, preferred_element_type=f32)


def _make_tables(latents, variance, posp, postgt, wlt, wl2, wpt, wp2):
    n_src = latents.shape[0]
    n_tgt = postgt.shape[0]
    return pl.pallas_call(
        _tables_body,
        out_shape=(
            jax.ShapeDtypeStruct((n_src, 2 * LATENT), jnp.float32),
            jax.ShapeDtypeStruct((n_tgt, 2 * LATENT), jnp.float32),
        ),
    )(latents, variance, posp, postgt, wlt, wl2, wpt, wp2)


_KNN_TB = 256


def _knn_body(yt_ref, xt_ref, idx_ref):
    y = yt_ref[...]
    xt = xt_ref[...]
    n_src = xt.shape[1]
    tb = y.shape[0]
    yy = jnp.sum(y * y, axis=1, keepdims=True)
    xx = jnp.sum(xt * xt, axis=0, keepdims=True)
    dot = jnp.dot(y, xt, preferred_element_type=jnp.float32)
    d = (yy + xx) - 2.0 * dot
    colf = lax.broadcasted_iota(jnp.int32, (tb, n_src), 1).astype(jnp.float32)
    big = jnp.float32(1e30)
    inf = jnp.float32(jnp.inf)
    cols = []
    for _ in range(K):
        m = jnp.min(d, axis=1, keepdims=True)
        sel = jnp.where(d <= m, colf, big)
        idxf = jnp.min(sel, axis=1, keepdims=True)
        cols.append(idxf)
        d = jnp.where(colf == idxf, inf, d)
    idx_ref[...] = jnp.concatenate(cols, axis=1).astype(jnp.int32)


def _knn(postgt_pad, pos_t):
    n_tgt = postgt_pad.shape[0]
    grid = n_tgt // _KNN_TB
    return pl.pallas_call(
        _knn_body,
        grid=(grid,),
        in_specs=[
            pl.BlockSpec((_KNN_TB, 8), lambda i: (i, 0)),
            pl.BlockSpec(pos_t.shape, lambda i: (0, 0)),
        ],
        out_specs=pl.BlockSpec((_KNN_TB, K), lambda i: (i, 0)),
        out_shape=jax.ShapeDtypeStruct((n_tgt, K), jnp.int32),
    )(postgt_pad, pos_t)


_SC_CHUNK = 128


def _sc_gather(table, idx_flat):
    n_rows = idx_flat.shape[0]
    width = table.shape[1]
    info = plsc.get_sparse_core_info()
    nw = info.num_cores * info.num_subcores
    per_w = n_rows // nw
    n_chunks = per_w // _SC_CHUNK
    mesh = plsc.VectorSubcoreMesh(core_axis_name="c", subcore_axis_name="s")

    @functools.partial(
        pl.kernel,
        out_type=jax.ShapeDtypeStruct((n_rows, width), jnp.float32),
        mesh=mesh,
        scratch_types=[
            pltpu.VMEM((_SC_CHUNK,), jnp.int32),
            pltpu.VMEM((_SC_CHUNK, width), jnp.float32),
            pltpu.SemaphoreType.DMA,
        ],
        compiler_params=pltpu.CompilerParams(use_tc_tiling_on_sc=True),
    )
    def gather_k(table_hbm, idx_hbm, out_hbm, idx_v, rows_v, sem):
        wid = lax.axis_index("s") * info.num_cores + lax.axis_index("c")
        base_w = wid * per_w

        def chunk(c, _):
            base = base_w + c * _SC_CHUNK
            pltpu.sync_copy(idx_hbm.at[pl.ds(base, _SC_CHUNK)], idx_v)
            pltpu.async_copy(table_hbm.at[idx_v], rows_v, sem).wait()
            pltpu.sync_copy(rows_v, out_hbm.at[pl.ds(base, _SC_CHUNK)])
            return _

        lax.fori_loop(0, n_chunks, chunk, None)

    return gather_k(table, idx_flat)


_MLP_BLK = 2048
_MLP_TGT = _MLP_BLK // K

_SQRT_2PI = 2.5066282746310002
_SQRT_2 = 1.4142135623730951


def _adf_relu(mean, var):
    std = jnp.sqrt(jnp.clip(var, 1e-12, None))
    div = mean / std
    pdf = jnp.exp(-0.5 * div * div) / _SQRT_2PI
    cdf = 0.5 * (1.0 + lax.erf(div / _SQRT_2))
    out_mean = mean * cdf + std * pdf
    out_var = (mean * mean + var) * cdf + mean * std * pdf \
        - out_mean * out_mean
    return out_mean, out_var + MIN_VAR


def _mlp_body(g_ref, t_ref, occ_ref, w1t_ref, w12_ref, w2t_ref, w22_ref,
              wot_ref, wo2_ref, bin_ref, b1_ref, b2_ref, bo_ref,
              pred_ref, psum_ref):
    f32 = jnp.float32
    g = g_ref[...]
    t = t_ref[...]
    tgt, width = t.shape
    blk = g.shape[0]
    lat = LATENT
    t_rep = jnp.reshape(
        jnp.broadcast_to(t[:, None, :], (tgt, K, width)), (blk, width))
    mean = g[:, :lat] + t_rep[:, :lat] + bin_ref[...]
    var = g[:, lat:] + t_rep[:, lat:] + MIN_VAR

    mean, var = _adf_relu(mean, var)
    mean = jnp.dot(mean, w1t_ref[...], preferred_element_type=f32) \
        + b1_ref[...]
    var = jnp.dot(var, w12_ref[...], preferred_element_type=f32) + MIN_VAR

    mean, var = _adf_relu(mean, var)
    mean = jnp.dot(mean, w2t_ref[...], preferred_element_type=f32) \
        + b2_ref[...]
    var = jnp.dot(var, w22_ref[...], preferred_element_type=f32) + MIN_VAR

    out_m = jnp.dot(mean, wot_ref[...], preferred_element_type=f32) \
        + bo_ref[...]
    out_v = jnp.dot(var, wo2_ref[...], preferred_element_type=f32) + MIN_VAR

    z = jnp.reshape(out_m[:, 0:1], (blk // 128, 128))
    pred_ref[...] = z
    vsum = jnp.sum(jnp.abs(out_v[:, 0:1]))
    occ = occ_ref[...]
    loss = jnp.maximum(z, 0.0) - z * occ + jnp.log1p(jnp.exp(-jnp.abs(z)))
    lsum = jnp.sum(loss)
    psum_ref[...] = jnp.concatenate(
        [jnp.reshape(vsum, (1, 1, 1)), jnp.reshape(lsum, (1, 1, 1))], axis=2)


def _mlp(g, t_tgt, occ_gt, w1t, w12, w2t, w22, wot, wo2, b_in, b1, b2, b_o):
    n_rows = g.shape[0]
    grid = n_rows // _MLP_BLK
    full = lambda a: pl.BlockSpec(a.shape, lambda i: (0,) * a.ndim)
    occ_gt = occ_gt.reshape(n_rows // 128, 128)
    pred, psum = pl.pallas_call(
        _mlp_body,
        grid=(grid,),
        in_specs=[
            pl.BlockSpec((_MLP_BLK, 2 * LATENT), lambda i: (i, 0)),
            pl.BlockSpec((_MLP_TGT, 2 * LATENT), lambda i: (i, 0)),
            pl.BlockSpec((_MLP_BLK // 128, 128), lambda i: (i, 0)),
            full(w1t), full(w12), full(w2t), full(w22), full(wot), full(wo2),
            full(b_in), full(b1), full(b2), full(b_o),
        ],
        out_specs=(
            pl.BlockSpec((_MLP_BLK // 128, 128), lambda i: (i, 0)),
            pl.BlockSpec((1, 1, 2), lambda i: (i, 0, 0)),
        ),
        out_shape=(
            jax.ShapeDtypeStruct((n_rows // 128, 128), jnp.float32),
            jax.ShapeDtypeStruct((grid, 1, 2), jnp.float32),
        ),
    )(g, t_tgt, occ_gt, w1t, w12, w2t, w22, wot, wo2, b_in, b1, b2, b_o)
    return pred, psum


def kernel(pos, batch, pos_non_manifold, pos_non_manifold_batch, latents,
           variance, occupancies, W_in, b_in, W1, b1, W2, b2, W_out, b_out,
           dropout_rate):
    n_src = pos.shape[0]
    n_tgt = pos_non_manifold.shape[0]
    f32 = jnp.float32

    w_lat = W_in[:, :LATENT]
    w_pos = W_in[:, LATENT:]
    wlt = w_lat.T
    wl2 = (w_lat * w_lat).T
    wpt = jnp.concatenate([w_pos.T, jnp.zeros((5, LATENT), f32)], axis=0)
    wp2 = jnp.concatenate([(w_pos * w_pos).T, jnp.zeros((5, LATENT), f32)],
                          axis=0)
    posp = jnp.pad(pos, ((0, 0), (0, 5)))
    postp = jnp.pad(pos_non_manifold, ((0, 0), (0, 5)))
    pos_t = posp.T

    t_src, t_tgt = _make_tables(latents, variance, posp, postp,
                                wlt, wl2, wpt, wp2)

    occ_gt = jnp.broadcast_to(occupancies[:, None],
                              (n_tgt, K)).reshape(-1)

    w1t = W1.T
    w12 = (W1 * W1).T
    w2t = W2.T
    w22 = (W2 * W2).T
    wot = W_out.T
    wo2 = (W_out * W_out).T
    weights = (w1t, w12, w2t, w22, wot, wo2, b_in.reshape(1, -1),
               b1.reshape(1, -1), b2.reshape(1, -1), b_out.reshape(1, -1))

    h = n_tgt // 2
    idx_a = _knn(postp[:h], pos_t)
    g_a = _sc_gather(t_src, idx_a.reshape(-1))
    idx_b = _knn(postp[h:], pos_t)
    g_b = _sc_gather(t_src, idx_b.reshape(-1))
    pred_a, psum_a = _mlp(g_a, t_tgt[:h], occ_gt[:h * K], *weights)
    pred_b, psum_b = _mlp(g_b, t_tgt[h:], occ_gt[h * K:], *weights)

    n_pairs = n_tgt * K
    predictions = jnp.concatenate([pred_a, pred_b]).reshape(n_pairs)
    aleatoric = (jnp.sum(psum_a[:, 0, 0]) + jnp.sum(psum_b[:, 0, 0])) / n_pairs
    recons = (jnp.sum(psum_a[:, 0, 1]) + jnp.sum(psum_b[:, 0, 1])) / n_pairs
    return (predictions, aleatoric, occ_gt, predictions, recons)
